# Initial kernel scaffold; baseline (speedup 1.0000x reference)
import jax
import jax.numpy as jnp
from jax.experimental import pallas as pl


def kernel(node_states, edge_index, W1, b1, W2, b2, U1, c1, U2, c2):
    raise NotImplementedError("write your pallas kernel here")


# speedup vs baseline: 2.9056x; 2.9056x over previous
import functools

import jax
import jax.numpy as jnp
from jax import lax
from jax.experimental import pallas as pl
from jax.experimental.pallas import tpu as pltpu
from jax.experimental.pallas import tpu_sc as plsc

N_NODES = 10000
N_EDGES = 320000
D = 128

NC = 2
NS = 16
NW = NC * NS
EPW = N_EDGES // NW
CHUNK = 80
NCHUNK = EPW // CHUNK
NPAD = 10240
ROWS_PER_TILE = NPAD // NS
ZROWS = 128
DROWS = NPAD // D

_sc_mesh = plsc.VectorSubcoreMesh(core_axis_name="c", subcore_axis_name="s")


@functools.partial(
    pl.kernel,
    out_type=[
        jax.ShapeDtypeStruct((NC, NPAD, D), jnp.float32),
        jax.ShapeDtypeStruct((NC, DROWS, D), jnp.float32),
    ],
    mesh=_sc_mesh,
    scratch_types=[
        pltpu.VMEM((CHUNK,), jnp.int32),
        pltpu.VMEM((CHUNK,), jnp.int32),
        pltpu.VMEM((CHUNK,), jnp.int32),
        pltpu.VMEM((CHUNK, D), jnp.float32),
        pltpu.VMEM((CHUNK, D), jnp.float32),
        pltpu.VMEM((CHUNK, D), jnp.float32),
        pltpu.VMEM((D,), jnp.float32),
        pltpu.VMEM((DROWS, D), jnp.float32),
        pltpu.VMEM_SHARED((NPAD, D), jnp.float32),
        pltpu.VMEM_SHARED((DROWS, D), jnp.float32),
        pltpu.SemaphoreType.DMA,
    ],
)
def _sc_edge_kernel(a_hbm, b_hbm, src_hbm, dst_hbm, b1_hbm, s_out, deg_out,
                    src_v, dst_v, ident_v, a_v, b_v, r_v, b1_v,
                    deg_v, s_sh, deg_sh, sem):
    c = lax.axis_index("c")
    s = lax.axis_index("s")
    w = s * NC + c

    zeros16 = jnp.zeros((16,), jnp.float32)
    ones16 = jnp.ones((16,), jnp.float32)
    iota16 = lax.iota(jnp.int32, 16)

    def zrow(i, _):
        for j in range(D // 16):
            r_v[i, pl.ds(j * 16, 16)] = zeros16
        return 0
    lax.fori_loop(0, CHUNK, zrow, 0)

    def zdeg(i, _):
        for j in range(D // 16):
            deg_v[i, pl.ds(j * 16, 16)] = zeros16
        return 0
    lax.fori_loop(0, DROWS, zdeg, 0)

    for k in range(CHUNK // 16):
        ident_v[pl.ds(k * 16, 16)] = iota16 + k * 16

    for k in range(ROWS_PER_TILE // CHUNK):
        pltpu.sync_copy(r_v, s_sh.at[pl.ds(s * ROWS_PER_TILE + k * CHUNK, CHUNK)])

    @pl.when(s == 0)
    def _():
        pltpu.sync_copy(r_v, deg_sh)

    pltpu.sync_copy(b1_hbm, b1_v)
    plsc.subcore_barrier()

    def chunk_body(ci, _):
        base = w * EPW + ci * CHUNK
        pltpu.sync_copy(src_hbm.at[pl.ds(base, CHUNK)], src_v)
        pltpu.sync_copy(dst_hbm.at[pl.ds(base, CHUNK)], dst_v)
        pltpu.async_copy(a_hbm.at[dst_v], a_v, sem).wait()
        pltpu.async_copy(b_hbm.at[src_v], b_v, sem).wait()

        def hist_body(k, _):
            dvec = dst_v[pl.ds(k * 16, 16)]
            for l in range(16):
                d = dvec[l]
                row = lax.shift_right_logical(d, 7)
                cb = lax.bitwise_and(d, 112)
                lane = lax.bitwise_and(d, 15)
                hv = deg_v[row, pl.ds(cb, 16)]
                deg_v[row, pl.ds(cb, 16)] = hv + jnp.where(
                    iota16 == lane, ones16, zeros16)
            return 0
        lax.fori_loop(0, CHUNK // 16, hist_body, 0)

        def edge_body(e, _):
            for j in range(D // 16):
                av = a_v[e, pl.ds(j * 16, 16)]
                bv = b_v[e, pl.ds(j * 16, 16)]
                b1v = b1_v[pl.ds(j * 16, 16)]
                r_v[e, pl.ds(j * 16, 16)] = jnp.maximum(av + bv + b1v, 0.0)
            return 0
        lax.fori_loop(0, CHUNK, edge_body, 0)

        pltpu.sync_copy(r_v, s_sh.at[dst_v], add=True)
        return 0
    lax.fori_loop(0, NCHUNK, chunk_body, 0)

    plsc.subcore_barrier()
    pltpu.sync_copy(deg_v, deg_sh.at[ident_v], add=True)
    plsc.subcore_barrier()

    row0 = s * ROWS_PER_TILE
    pltpu.sync_copy(s_sh.at[pl.ds(row0, ROWS_PER_TILE)],
                    s_out.at[c, pl.ds(row0, ROWS_PER_TILE)])

    @pl.when(s == 0)
    def _():
        pltpu.sync_copy(deg_sh, deg_out.at[c])


_ROWS_BLK = 1280


def _proj_body(ns_ref, w1a_ref, w1b_ref, a_ref, b_ref):
    x = ns_ref[...]
    a_ref[...] = jnp.dot(x, w1a_ref[...], preferred_element_type=jnp.float32)
    b_ref[...] = jnp.dot(x, w1b_ref[...], preferred_element_type=jnp.float32)


def _proj(node_states, w1a, w1b):
    blk = 1000
    grid = (N_NODES // blk,)
    return pl.pallas_call(
        _proj_body,
        grid=grid,
        in_specs=[
            pl.BlockSpec((blk, D), lambda i: (i, 0)),
            pl.BlockSpec((D, D), lambda i: (0, 0)),
            pl.BlockSpec((D, D), lambda i: (0, 0)),
        ],
        out_specs=[
            pl.BlockSpec((blk, D), lambda i: (i, 0)),
            pl.BlockSpec((blk, D), lambda i: (i, 0)),
        ],
        out_shape=[
            jax.ShapeDtypeStruct((N_NODES, D), jnp.float32),
            jax.ShapeDtypeStruct((N_NODES, D), jnp.float32),
        ],
    )(node_states, w1a, w1b)


def _update_body(s0_ref, s1_ref, d0_ref, d1_ref, ns_ref, w2_ref, b2_ref,
                 u1a_ref, u1b_ref, c1_ref, u2_ref, c2_ref, out_ref):
    sm = s0_ref[...] + s1_ref[...]
    deg = d0_ref[...] + d1_ref[...]
    m_agg = (jnp.dot(sm, w2_ref[...], preferred_element_type=jnp.float32)
             + deg * b2_ref[...])
    pre = (jnp.dot(ns_ref[...], u1a_ref[...], preferred_element_type=jnp.float32)
           + jnp.dot(m_agg, u1b_ref[...], preferred_element_type=jnp.float32)
           + c1_ref[...])
    h = jnp.maximum(pre, 0.0)
    out_ref[...] = (jnp.dot(h, u2_ref[...], preferred_element_type=jnp.float32)
                    + c2_ref[...])


def _update(s0, s1, d0, d1, ns_pad, w2, b2, u1a, u1b, c1, u2, c2):
    grid = (NPAD // _ROWS_BLK,)
    full = lambda r, c_: pl.BlockSpec((r, c_), lambda i: (0, 0))
    return pl.pallas_call(
        _update_body,
        grid=grid,
        in_specs=[
            pl.BlockSpec((_ROWS_BLK, D), lambda i: (i, 0)),
            pl.BlockSpec((_ROWS_BLK, D), lambda i: (i, 0)),
            pl.BlockSpec((_ROWS_BLK, 1), lambda i: (i, 0)),
            pl.BlockSpec((_ROWS_BLK, 1), lambda i: (i, 0)),
            pl.BlockSpec((_ROWS_BLK, D), lambda i: (i, 0)),
            full(D, D), full(1, D), full(D, D), full(D, D),
            full(1, D), full(D, D), full(1, D),
        ],
        out_specs=pl.BlockSpec((_ROWS_BLK, D), lambda i: (i, 0)),
        out_shape=jax.ShapeDtypeStruct((NPAD, D), jnp.float32),
    )(s0, s1, d0, d1, ns_pad, w2, b2, u1a, u1b, c1, u2, c2)


def kernel(node_states, edge_index, W1, b1, W2, b2, U1, c1, U2, c2):
    src = edge_index[0].astype(jnp.int32)
    dst = edge_index[1].astype(jnp.int32)
    a, b = _proj(node_states, W1[:D], W1[D:])
    s_parts, deg_parts = _sc_edge_kernel(a, b, src, dst, b1)
    deg_cols = deg_parts.reshape(NC, NPAD, 1)
    ns_pad = jnp.pad(node_states, ((0, NPAD - N_NODES), (0, 0)))
    out = _update(s_parts[0], s_parts[1], deg_cols[0], deg_cols[1], ns_pad,
                  W2, b2.reshape(1, D), U1[:D], U1[D:],
                  c1.reshape(1, D), U2, c2.reshape(1, D))
    return out[:N_NODES]


# speedup vs baseline: 4.6100x; 1.5866x over previous
import functools

import jax
import jax.numpy as jnp
from jax import lax
from jax.experimental import pallas as pl
from jax.experimental.pallas import tpu as pltpu
from jax.experimental.pallas import tpu_sc as plsc

N_NODES = 10000
N_EDGES = 320000
D = 128

NC = 2
NS = 16
NW = NC * NS
CHUNK = 64
TOTCH = N_EDGES // CHUNK
NITER = 158
NPAD = 10240
ROWS_PER_TILE = NPAD // NS
DROWS = NPAD // D

_sc_mesh = plsc.VectorSubcoreMesh(core_axis_name="c", subcore_axis_name="s")


@functools.partial(
    pl.kernel,
    out_type=[
        jax.ShapeDtypeStruct((NC, NPAD, D), jnp.float32),
        jax.ShapeDtypeStruct((NC, DROWS, D), jnp.float32),
    ],
    mesh=_sc_mesh,
    scratch_types=[
        pltpu.VMEM((2, CHUNK), jnp.int32),
        pltpu.VMEM((2, CHUNK), jnp.int32),
        pltpu.VMEM((2, CHUNK), jnp.int32),
        pltpu.VMEM((2, CHUNK, D), jnp.float32),
        pltpu.VMEM((2, CHUNK, D), jnp.float32),
        pltpu.VMEM((D,), jnp.float32),
        pltpu.VMEM((DROWS,), jnp.int32),
        pltpu.VMEM((DROWS, D), jnp.float32),
        pltpu.VMEM_SHARED((NPAD, D), jnp.float32),
        pltpu.VMEM_SHARED((DROWS, D), jnp.float32),
        pltpu.SemaphoreType.DMA,
        pltpu.SemaphoreType.DMA,
        pltpu.SemaphoreType.DMA,
        pltpu.SemaphoreType.DMA,
        pltpu.SemaphoreType.DMA,
        pltpu.SemaphoreType.DMA,
    ],
)
def _sc_edge_kernel(a_hbm, b_hbm, src_hbm, dst_hbm, b1_hbm, s_out, deg_out,
                    src_v, dst_v, dsc_v, a_v, b_v, b1_v, ident_v, deg_v,
                    s_sh, deg_sh,
                    sem_i0, sem_i1, sem_g0, sem_g1, sem_s0, sem_s1):
    c = lax.axis_index("c")
    s = lax.axis_index("s")
    w = s * NC + c
    sem_i = (sem_i0, sem_i1)
    sem_g = (sem_g0, sem_g1)
    sem_s = (sem_s0, sem_s1)

    zeros16 = jnp.zeros((16,), jnp.float32)
    ones16 = jnp.ones((16,), jnp.float32)
    iota16 = lax.iota(jnp.int32, 16)

    def chunk_id(ci):
        return w + ci * NW

    def base_of(ci):
        return chunk_id(ci) * CHUNK

    def issue_idx(ci, b):
        @pl.when(chunk_id(ci) < TOTCH)
        def _():
            base = base_of(ci)
            pltpu.async_copy(src_hbm.at[pl.ds(base, CHUNK)], src_v.at[b],
                             sem_i[b])
            pltpu.async_copy(dst_hbm.at[pl.ds(base, CHUNK)], dst_v.at[b],
                             sem_i[b])

    def drain_idx(b):
        pltpu.make_async_copy(src_hbm.at[pl.ds(0, CHUNK)], src_v.at[b],
                              sem_i[b]).wait()
        pltpu.make_async_copy(dst_hbm.at[pl.ds(0, CHUNK)], dst_v.at[b],
                              sem_i[b]).wait()

    def issue_gather(b):
        pltpu.async_copy(a_hbm.at[dst_v.at[b]], a_v.at[b], sem_g[b])
        pltpu.async_copy(b_hbm.at[src_v.at[b]], b_v.at[b], sem_g[b])

    def drain_gather(b):
        pltpu.make_async_copy(a_hbm.at[pl.ds(0, CHUNK)], a_v.at[b],
                              sem_g[b]).wait()
        pltpu.make_async_copy(b_hbm.at[pl.ds(0, CHUNK)], b_v.at[b],
                              sem_g[b]).wait()

    def drain_scatter(b):
        pltpu.make_async_copy(a_hbm.at[pl.ds(0, CHUNK)], a_v.at[b],
                              sem_s[b]).wait()

    def zrow(i, _):
        for j in range(D // 16):
            a_v[0, i, pl.ds(j * 16, 16)] = zeros16
        return 0
    lax.fori_loop(0, CHUNK, zrow, 0)

    def zdeg(i, _):
        for j in range(D // 16):
            deg_v[i, pl.ds(j * 16, 16)] = zeros16
        return 0
    lax.fori_loop(0, DROWS, zdeg, 0)

    for k in range(ROWS_PER_TILE // CHUNK):
        pltpu.sync_copy(a_v.at[0],
                        s_sh.at[pl.ds(s * ROWS_PER_TILE + k * CHUNK, CHUNK)])

    @pl.when(s == 0)
    def _():
        pltpu.sync_copy(a_v.at[0], deg_sh.at[pl.ds(0, CHUNK)])

    @pl.when(s == 1)
    def _():
        pltpu.sync_copy(a_v.at[0].at[pl.ds(0, DROWS - CHUNK)],
                        deg_sh.at[pl.ds(CHUNK, DROWS - CHUNK)])

    for k in range(DROWS // 16):
        ident_v[pl.ds(k * 16, 16)] = iota16 + k * 16

    pltpu.sync_copy(b1_hbm, b1_v)
    plsc.subcore_barrier()

    issue_idx(0, 0)
    issue_idx(1, 1)
    drain_idx(0)
    issue_gather(0)

    def pair_body(kk, _):
        for b in (0, 1):
            ci = 2 * kk + b
            o = 1 - b
            valid_c = chunk_id(ci) < TOTCH

            @pl.when(valid_c)
            def _(b=b, ci=ci):
                drain_gather(b)

                for g in range(CHUNK // 16):
                    dsc_v[b, pl.ds(g * 16, 16)] = dst_v[b, pl.ds(g * 16, 16)]

            issue_idx(ci + 2, b)

            @pl.when(jnp.logical_and(ci >= 1, chunk_id(ci - 1) < TOTCH))
            def _(o=o):
                drain_scatter(o)

            @pl.when(chunk_id(ci + 1) < TOTCH)
            def _(o=o):
                drain_idx(o)
                issue_gather(o)

            @pl.when(valid_c)
            def _(b=b):
                def hist_body(k, _):
                    dvec = dsc_v[b, pl.ds(k * 16, 16)]
                    for l in range(16):
                        d = dvec[l]
                        row = lax.shift_right_logical(d, 7)
                        cb = lax.bitwise_and(d, 112)
                        lane = lax.bitwise_and(d, 15)
                        hv = deg_v[row, pl.ds(cb, 16)]
                        deg_v[row, pl.ds(cb, 16)] = hv + jnp.where(
                            iota16 == lane, ones16, zeros16)
                    return 0
                lax.fori_loop(0, CHUNK // 16, hist_body, 0)

                def edge_body(e, _):
                    for j in range(D // 16):
                        av = a_v[b, e, pl.ds(j * 16, 16)]
                        bv = b_v[b, e, pl.ds(j * 16, 16)]
                        b1v = b1_v[pl.ds(j * 16, 16)]
                        a_v[b, e, pl.ds(j * 16, 16)] = jnp.maximum(
                            av + bv + b1v, 0.0)
                    return 0
                lax.fori_loop(0, CHUNK, edge_body, 0)

                pltpu.async_copy(a_v.at[b], s_sh.at[dsc_v.at[b]], sem_s[b],
                                 add=True)
        return 0
    lax.fori_loop(0, NITER // 2, pair_body, 0)

    plsc.subcore_barrier()
    pltpu.sync_copy(deg_v, deg_sh.at[ident_v], add=True)
    plsc.subcore_barrier()

    row0 = s * ROWS_PER_TILE
    pltpu.sync_copy(s_sh.at[pl.ds(row0, ROWS_PER_TILE)],
                    s_out.at[c, pl.ds(row0, ROWS_PER_TILE)])

    @pl.when(s == 0)
    def _():
        pltpu.sync_copy(deg_sh, deg_out.at[c])


_ROWS_BLK = 1280


def _proj_body(ns_ref, w1a_ref, w1b_ref, a_ref, b_ref):
    x = ns_ref[...]
    a_ref[...] = jnp.dot(x, w1a_ref[...], preferred_element_type=jnp.float32)
    b_ref[...] = jnp.dot(x, w1b_ref[...], preferred_element_type=jnp.float32)


def _proj(node_states, w1a, w1b):
    blk = 1000
    grid = (N_NODES // blk,)
    return pl.pallas_call(
        _proj_body,
        grid=grid,
        in_specs=[
            pl.BlockSpec((blk, D), lambda i: (i, 0)),
            pl.BlockSpec((D, D), lambda i: (0, 0)),
            pl.BlockSpec((D, D), lambda i: (0, 0)),
        ],
        out_specs=[
            pl.BlockSpec((blk, D), lambda i: (i, 0)),
            pl.BlockSpec((blk, D), lambda i: (i, 0)),
        ],
        out_shape=[
            jax.ShapeDtypeStruct((N_NODES, D), jnp.float32),
            jax.ShapeDtypeStruct((N_NODES, D), jnp.float32),
        ],
    )(node_states, w1a, w1b)


def _update_body(s0_ref, s1_ref, d0_ref, d1_ref, ns_ref, w2_ref, b2_ref,
                 u1a_ref, u1b_ref, c1_ref, u2_ref, c2_ref, out_ref):
    sm = s0_ref[...] + s1_ref[...]
    deg = d0_ref[...] + d1_ref[...]
    m_agg = (jnp.dot(sm, w2_ref[...], preferred_element_type=jnp.float32)
             + deg * b2_ref[...])
    pre = (jnp.dot(ns_ref[...], u1a_ref[...], preferred_element_type=jnp.float32)
           + jnp.dot(m_agg, u1b_ref[...], preferred_element_type=jnp.float32)
           + c1_ref[...])
    h = jnp.maximum(pre, 0.0)
    out_ref[...] = (jnp.dot(h, u2_ref[...], preferred_element_type=jnp.float32)
                    + c2_ref[...])


def _update(s0, s1, d0, d1, ns_pad, w2, b2, u1a, u1b, c1, u2, c2):
    grid = (NPAD // _ROWS_BLK,)
    full = lambda r, c_: pl.BlockSpec((r, c_), lambda i: (0, 0))
    return pl.pallas_call(
        _update_body,
        grid=grid,
        in_specs=[
            pl.BlockSpec((_ROWS_BLK, D), lambda i: (i, 0)),
            pl.BlockSpec((_ROWS_BLK, D), lambda i: (i, 0)),
            pl.BlockSpec((_ROWS_BLK, 1), lambda i: (i, 0)),
            pl.BlockSpec((_ROWS_BLK, 1), lambda i: (i, 0)),
            pl.BlockSpec((_ROWS_BLK, D), lambda i: (i, 0)),
            full(D, D), full(1, D), full(D, D), full(D, D),
            full(1, D), full(D, D), full(1, D),
        ],
        out_specs=pl.BlockSpec((_ROWS_BLK, D), lambda i: (i, 0)),
        out_shape=jax.ShapeDtypeStruct((NPAD, D), jnp.float32),
    )(s0, s1, d0, d1, ns_pad, w2, b2, u1a, u1b, c1, u2, c2)


def kernel(node_states, edge_index, W1, b1, W2, b2, U1, c1, U2, c2):
    src = edge_index[0].astype(jnp.int32)
    dst = edge_index[1].astype(jnp.int32)
    a, b = _proj(node_states, W1[:D], W1[D:])
    s_parts, deg_parts = _sc_edge_kernel(a, b, src, dst, b1)
    deg_cols = deg_parts.reshape(NC, NPAD, 1)
    ns_pad = jnp.pad(node_states, ((0, NPAD - N_NODES), (0, 0)))
    out = _update(s_parts[0], s_parts[1], deg_cols[0], deg_cols[1], ns_pad,
                  W2, b2.reshape(1, D), U1[:D], U1[D:],
                  c1.reshape(1, D), U2, c2.reshape(1, D))
    return out[:N_NODES]


# speedup vs baseline: 9.0250x; 1.9577x over previous
import functools

import jax
import jax.numpy as jnp
from jax import lax
from jax.experimental import pallas as pl
from jax.experimental.pallas import tpu as pltpu
from jax.experimental.pallas import tpu_sc as plsc

N_NODES = 10000
N_EDGES = 320000
D = 128

NC = 2
NS = 16
NW = NC * NS
CHUNK = 64
TOTCH = N_EDGES // CHUNK
NITER = 158
NPAD = 10240
ROWS_PER_TILE = NPAD // NS
DROWS = NPAD // D

_sc_mesh = plsc.VectorSubcoreMesh(core_axis_name="c", subcore_axis_name="s")


@functools.partial(
    pl.kernel,
    out_type=[
        jax.ShapeDtypeStruct((NC, NPAD, D), jnp.float32),
        jax.ShapeDtypeStruct((NC, DROWS, D), jnp.float32),
    ],
    mesh=_sc_mesh,
    scratch_types=[
        pltpu.VMEM((2, CHUNK), jnp.int32),
        pltpu.VMEM((2, CHUNK), jnp.int32),
        pltpu.VMEM((2, CHUNK), jnp.int32),
        pltpu.VMEM((2, CHUNK, D), jnp.float32),
        pltpu.VMEM((2, CHUNK, D), jnp.float32),
        pltpu.VMEM((DROWS,), jnp.int32),
        pltpu.VMEM((DROWS, D), jnp.float32),
        pltpu.VMEM_SHARED((NPAD, D), jnp.float32),
        pltpu.VMEM_SHARED((DROWS, D), jnp.float32),
        pltpu.SemaphoreType.DMA,
        pltpu.SemaphoreType.DMA,
        pltpu.SemaphoreType.DMA,
        pltpu.SemaphoreType.DMA,
        pltpu.SemaphoreType.DMA,
        pltpu.SemaphoreType.DMA,
    ],
)
def _sc_edge_kernel(a_hbm, b_hbm, src_hbm, dst_hbm, s_out, deg_out,
                    src_v, dst_v, dsc_v, a_v, b_v, ident_v, deg_v,
                    s_sh, deg_sh,
                    sem_i0, sem_i1, sem_g0, sem_g1, sem_s0, sem_s1):
    c = lax.axis_index("c")
    s = lax.axis_index("s")
    w = s * NC + c
    sem_i = (sem_i0, sem_i1)
    sem_g = (sem_g0, sem_g1)
    sem_s = (sem_s0, sem_s1)

    zeros16 = jnp.zeros((16,), jnp.float32)
    ones16 = jnp.ones((16,), jnp.float32)
    iota16 = lax.iota(jnp.int32, 16)

    def chunk_id(ci):
        return w + ci * NW

    def base_of(ci):
        return chunk_id(ci) * CHUNK

    def issue_idx(ci, b):
        @pl.when(chunk_id(ci) < TOTCH)
        def _():
            base = base_of(ci)
            pltpu.async_copy(src_hbm.at[pl.ds(base, CHUNK)], src_v.at[b],
                             sem_i[b])
            pltpu.async_copy(dst_hbm.at[pl.ds(base, CHUNK)], dst_v.at[b],
                             sem_i[b])

    def drain_idx(b):
        pltpu.make_async_copy(src_hbm.at[pl.ds(0, CHUNK)], src_v.at[b],
                              sem_i[b]).wait()
        pltpu.make_async_copy(dst_hbm.at[pl.ds(0, CHUNK)], dst_v.at[b],
                              sem_i[b]).wait()

    def issue_gather(b):
        pltpu.async_copy(a_hbm.at[dst_v.at[b]], a_v.at[b], sem_g[b])
        pltpu.async_copy(b_hbm.at[src_v.at[b]], b_v.at[b], sem_g[b])

    def drain_gather(b):
        pltpu.make_async_copy(a_hbm.at[pl.ds(0, CHUNK)], a_v.at[b],
                              sem_g[b]).wait()
        pltpu.make_async_copy(b_hbm.at[pl.ds(0, CHUNK)], b_v.at[b],
                              sem_g[b]).wait()

    def drain_scatter(b):
        pltpu.make_async_copy(a_hbm.at[pl.ds(0, CHUNK)], a_v.at[b],
                              sem_s[b]).wait()

    def zrow(i, _):
        for j in range(D // 16):
            a_v[0, i, pl.ds(j * 16, 16)] = zeros16
        return 0
    lax.fori_loop(0, CHUNK, zrow, 0)

    def zdeg(i, _):
        for j in range(D // 16):
            deg_v[i, pl.ds(j * 16, 16)] = zeros16
        return 0
    lax.fori_loop(0, DROWS, zdeg, 0)

    for k in range(ROWS_PER_TILE // CHUNK):
        pltpu.sync_copy(a_v.at[0],
                        s_sh.at[pl.ds(s * ROWS_PER_TILE + k * CHUNK, CHUNK)])

    @pl.when(s == 0)
    def _():
        pltpu.sync_copy(a_v.at[0], deg_sh.at[pl.ds(0, CHUNK)])

    @pl.when(s == 1)
    def _():
        pltpu.sync_copy(a_v.at[0].at[pl.ds(0, DROWS - CHUNK)],
                        deg_sh.at[pl.ds(CHUNK, DROWS - CHUNK)])

    for k in range(DROWS // 16):
        ident_v[pl.ds(k * 16, 16)] = iota16 + k * 16

    plsc.subcore_barrier()

    issue_idx(0, 0)
    issue_idx(1, 1)
    drain_idx(0)
    issue_gather(0)

    def pair_body(kk, _):
        for b in (0, 1):
            ci = 2 * kk + b
            o = 1 - b
            valid_c = chunk_id(ci) < TOTCH

            @pl.when(valid_c)
            def _(b=b, ci=ci):
                drain_gather(b)

                for g in range(CHUNK // 16):
                    dsc_v[b, pl.ds(g * 16, 16)] = dst_v[b, pl.ds(g * 16, 16)]

            issue_idx(ci + 2, b)

            @pl.when(jnp.logical_and(ci >= 1, chunk_id(ci - 1) < TOTCH))
            def _(o=o):
                drain_scatter(o)

            @pl.when(chunk_id(ci + 1) < TOTCH)
            def _(o=o):
                drain_idx(o)
                issue_gather(o)

            @pl.when(valid_c)
            def _(b=b):
                def hist_body(k, _):
                    dvec = dsc_v[b, pl.ds(k * 16, 16)]
                    for l in range(16):
                        d = dvec[l]
                        row = lax.shift_right_logical(d, 7)
                        cb = lax.bitwise_and(d, 112)
                        lane = lax.bitwise_and(d, 15)
                        hv = deg_v[row, pl.ds(cb, 16)]
                        deg_v[row, pl.ds(cb, 16)] = hv + jnp.where(
                            iota16 == lane, ones16, zeros16)
                    return 0
                lax.fori_loop(0, CHUNK // 16, hist_body, 0)

                @plsc.parallel_loop(0, CHUNK, unroll=4)
                def _(e):
                    for j in range(D // 16):
                        av = a_v[b, e, pl.ds(j * 16, 16)]
                        bv = b_v[b, e, pl.ds(j * 16, 16)]
                        a_v[b, e, pl.ds(j * 16, 16)] = jnp.maximum(
                            av + bv, 0.0)

                pltpu.async_copy(a_v.at[b], s_sh.at[dsc_v.at[b]], sem_s[b],
                                 add=True)
        return 0
    lax.fori_loop(0, NITER // 2, pair_body, 0)

    plsc.subcore_barrier()
    pltpu.sync_copy(deg_v, deg_sh.at[ident_v], add=True)
    plsc.subcore_barrier()

    row0 = s * ROWS_PER_TILE
    pltpu.sync_copy(s_sh.at[pl.ds(row0, ROWS_PER_TILE)],
                    s_out.at[c, pl.ds(row0, ROWS_PER_TILE)])

    @pl.when(s == 0)
    def _():
        pltpu.sync_copy(deg_sh, deg_out.at[c])


_ROWS_BLK = 1280


def _proj_body(ns_ref, w1a_ref, w1b_ref, b1_ref, a_ref, b_ref):
    x = ns_ref[...]
    a_ref[...] = (jnp.dot(x, w1a_ref[...], preferred_element_type=jnp.float32)
                  + b1_ref[...])
    b_ref[...] = jnp.dot(x, w1b_ref[...], preferred_element_type=jnp.float32)


def _proj(node_states, w1a, w1b, b1):
    blk = 1000
    grid = (N_NODES // blk,)
    return pl.pallas_call(
        _proj_body,
        grid=grid,
        in_specs=[
            pl.BlockSpec((blk, D), lambda i: (i, 0)),
            pl.BlockSpec((D, D), lambda i: (0, 0)),
            pl.BlockSpec((D, D), lambda i: (0, 0)),
            pl.BlockSpec((1, D), lambda i: (0, 0)),
        ],
        out_specs=[
            pl.BlockSpec((blk, D), lambda i: (i, 0)),
            pl.BlockSpec((blk, D), lambda i: (i, 0)),
        ],
        out_shape=[
            jax.ShapeDtypeStruct((N_NODES, D), jnp.float32),
            jax.ShapeDtypeStruct((N_NODES, D), jnp.float32),
        ],
    )(node_states, w1a, w1b, b1)


def _update_body(s0_ref, s1_ref, d0_ref, d1_ref, ns_ref, w2_ref, b2_ref,
                 u1a_ref, u1b_ref, c1_ref, u2_ref, c2_ref, out_ref):
    sm = s0_ref[...] + s1_ref[...]
    deg = d0_ref[...] + d1_ref[...]
    m_agg = (jnp.dot(sm, w2_ref[...], preferred_element_type=jnp.float32)
             + deg * b2_ref[...])
    pre = (jnp.dot(ns_ref[...], u1a_ref[...], preferred_element_type=jnp.float32)
           + jnp.dot(m_agg, u1b_ref[...], preferred_element_type=jnp.float32)
           + c1_ref[...])
    h = jnp.maximum(pre, 0.0)
    out_ref[...] = (jnp.dot(h, u2_ref[...], preferred_element_type=jnp.float32)
                    + c2_ref[...])


def _update(s0, s1, d0, d1, ns_pad, w2, b2, u1a, u1b, c1, u2, c2):
    grid = (NPAD // _ROWS_BLK,)
    full = lambda r, c_: pl.BlockSpec((r, c_), lambda i: (0, 0))
    return pl.pallas_call(
        _update_body,
        grid=grid,
        in_specs=[
            pl.BlockSpec((_ROWS_BLK, D), lambda i: (i, 0)),
            pl.BlockSpec((_ROWS_BLK, D), lambda i: (i, 0)),
            pl.BlockSpec((_ROWS_BLK, 1), lambda i: (i, 0)),
            pl.BlockSpec((_ROWS_BLK, 1), lambda i: (i, 0)),
            pl.BlockSpec((_ROWS_BLK, D), lambda i: (i, 0)),
            full(D, D), full(1, D), full(D, D), full(D, D),
            full(1, D), full(D, D), full(1, D),
        ],
        out_specs=pl.BlockSpec((_ROWS_BLK, D), lambda i: (i, 0)),
        out_shape=jax.ShapeDtypeStruct((NPAD, D), jnp.float32),
    )(s0, s1, d0, d1, ns_pad, w2, b2, u1a, u1b, c1, u2, c2)


def kernel(node_states, edge_index, W1, b1, W2, b2, U1, c1, U2, c2):
    src = edge_index[0].astype(jnp.int32)
    dst = edge_index[1].astype(jnp.int32)
    a, b = _proj(node_states, W1[:D], W1[D:], b1.reshape(1, D))
    s_parts, deg_parts = _sc_edge_kernel(a, b, src, dst)
    deg_cols = deg_parts.reshape(NC, NPAD, 1)
    ns_pad = jnp.pad(node_states, ((0, NPAD - N_NODES), (0, 0)))
    out = _update(s_parts[0], s_parts[1], deg_cols[0], deg_cols[1], ns_pad,
                  W2, b2.reshape(1, D), U1[:D], U1[D:],
                  c1.reshape(1, D), U2, c2.reshape(1, D))
    return out[:N_NODES]
